# baseline (speedup 1.0000x reference)
import jax
import jax.numpy as jnp
from jax import lax
from jax.experimental import pallas as pl
from jax.experimental.pallas import tpu as pltpu
from jax.experimental.pallas import tpu_sc as plsc

N = 10000
D = 128
T = 8
E = 320000

NC, NS = 2, 16
NW = NC * NS
CHUNK = 64
N_P = 10112
ROWS_PER_TILE = N_P // NS
BAND = 320
E_P = NS * BAND * CHUNK
SLAB = 16
PAIRS = SLAB // 2
CH0 = 80
CH1 = BAND - CH0

_mesh = plsc.VectorSubcoreMesh(core_axis_name="c", subcore_axis_name="s")

_pieces = []
_off = 0
while _off < ROWS_PER_TILE:
    _sz = min(CHUNK, ROWS_PER_TILE - _off)
    _pieces.append((_off, _sz))
    _off += _sz


def _sc_shell(body_mid):
    def body(acc_sh, buf_v, zacc_hbm, acc_out, c, s):
        base = s * ROWS_PER_TILE
        pltpu.sync_copy(zacc_hbm, buf_v)
        for o, z in _pieces:
            pltpu.sync_copy(buf_v.at[pl.ds(0, z), :],
                            acc_sh.at[pl.ds(base + o, z), :])
        plsc.subcore_barrier()
        body_mid()
        plsc.subcore_barrier()
        for o, z in _pieces:
            pltpu.sync_copy(acc_sh.at[pl.ds(base + o, z), :],
                            buf_v.at[pl.ds(0, z), :])
            pltpu.sync_copy(buf_v.at[pl.ds(0, z), :],
                            acc_out.at[c, pl.ds(base + o, z), :])

    return body


def _make_sc_agg():
    scratch = [
        pltpu.VMEM((SLAB, CHUNK), jnp.int32),
        pltpu.VMEM((SLAB, CHUNK), jnp.int32),
        pltpu.VMEM((CHUNK, D), jnp.float32),
        pltpu.VMEM((CHUNK, D), jnp.float32),
        pltpu.VMEM_SHARED((N_P, D), jnp.float32),
        pltpu.SemaphoreType.DMA,
        pltpu.SemaphoreType.DMA,
        pltpu.SemaphoreType.DMA,
        pltpu.SemaphoreType.DMA,
    ]

    def body(x_hbm, srcs_hbm, dsts_hbm, zacc_hbm, acc_out,
             src_v, dst_v, rows0, rows1, acc_sh, sg0, sg1, ss0, ss1):
        c = lax.axis_index("c")
        s = lax.axis_index("s")

        def wait_gather(buf, sem):
            pltpu.make_async_copy(x_hbm.at[src_v.at[0]], buf, sem).wait()

        def wait_scatter(buf, sem):
            pltpu.make_async_copy(buf, acc_sh.at[dst_v.at[0]], sem).wait()

        def run_chunks(start, n_chunks):
            def pair_step(t, carry):
                a = 2 * t
                b = a + 1

                @pl.when(t > 0)
                def _():
                    wait_scatter(rows1, ss1)

                wait_gather(rows0, sg0)
                pltpu.async_copy(x_hbm.at[src_v.at[b]], rows1, sg1)
                pltpu.async_copy(rows0, acc_sh.at[dst_v.at[a]], ss0, add=True)
                wait_gather(rows1, sg1)
                wait_scatter(rows0, ss0)

                @pl.when(t < PAIRS - 1)
                def _():
                    pltpu.async_copy(x_hbm.at[src_v.at[a + 2]], rows0, sg0)

                pltpu.async_copy(rows1, acc_sh.at[dst_v.at[b]], ss1, add=True)
                return carry

            def slab_step(g, carry):
                off = start + g * SLAB
                pltpu.sync_copy(srcs_hbm.at[s, pl.ds(off, SLAB), :], src_v)
                pltpu.sync_copy(dsts_hbm.at[s, pl.ds(off, SLAB), :], dst_v)
                pltpu.async_copy(x_hbm.at[src_v.at[0]], rows0, sg0)
                lax.fori_loop(0, PAIRS, pair_step, 0)
                wait_scatter(rows1, ss1)
                return carry

            lax.fori_loop(0, n_chunks // SLAB, slab_step, 0)

        def mid():
            @pl.when(c == 0)
            def _():
                run_chunks(0, CH0)

            @pl.when(c == 1)
            def _():
                run_chunks(CH0, CH1)

        _sc_shell(mid)(acc_sh, rows0, zacc_hbm, acc_out, c, s)

    return pl.kernel(body,
                     out_type=jax.ShapeDtypeStruct((NC, N_P, D), jnp.float32),
                     mesh=_mesh, scratch_types=scratch)


def _make_sc_deg():
    GRP = 8
    scratch = [
        pltpu.VMEM((SLAB, CHUNK), jnp.int32),
        pltpu.VMEM((CHUNK, D), jnp.float32),
        pltpu.VMEM_SHARED((N_P, D), jnp.float32),
        pltpu.SemaphoreType.DMA,
    ]

    def body(dsts_hbm, zacc_hbm, ones_hbm, acc_out, dst_v, ones_v, acc_sh, sem):
        c = lax.axis_index("c")
        s = lax.axis_index("s")

        def mid():
            pltpu.sync_copy(ones_hbm, ones_v)

            def grp_step(q, carry):
                for j in range(GRP):
                    pltpu.async_copy(ones_v, acc_sh.at[dst_v.at[q * GRP + j]],
                                     sem, add=True)
                for j in range(GRP):
                    pltpu.make_async_copy(ones_v, acc_sh.at[dst_v.at[0]],
                                          sem).wait()
                return carry

            def slab_step(g, carry):
                off = c * (BAND // 2) + g * SLAB
                pltpu.sync_copy(dsts_hbm.at[s, pl.ds(off, SLAB), :], dst_v)
                lax.fori_loop(0, SLAB // GRP, grp_step, 0)
                return carry

            lax.fori_loop(0, (BAND // 2) // SLAB, slab_step, 0)

        _sc_shell(mid)(acc_sh, ones_v, zacc_hbm, acc_out, c, s)

    return pl.kernel(body,
                     out_type=jax.ShapeDtypeStruct((NC, N_P, D), jnp.float32),
                     mesh=_mesh, scratch_types=scratch)


_sc_agg = _make_sc_agg()
_sc_deg = _make_sc_deg()

_B = 2528


def _make_tc_linear(relu):
    def body(p0_ref, p1_ref, d0_ref, d1_ref, tp_ref, w_ref, b_ref, o_ref):
        deg = d0_ref[:, 0:1] + d1_ref[:, 0:1]
        deg = jnp.maximum(deg, 1.0)
        agg = (p0_ref[...] + p1_ref[...]) / deg
        t = tp_ref[...]
        acc = jnp.zeros((_B, D), jnp.float32)
        for k in range(T):
            h = jnp.dot(agg, w_ref[k], preferred_element_type=jnp.float32)
            h = h + b_ref[k][None, :]
            acc = jnp.where(t == k, h, acc)
        if relu:
            acc = jnp.maximum(acc, 0.0)
        o_ref[...] = acc

    return pl.pallas_call(
        body,
        grid=(N_P // _B,),
        in_specs=[
            pl.BlockSpec((_B, D), lambda i: (i, 0)),
            pl.BlockSpec((_B, D), lambda i: (i, 0)),
            pl.BlockSpec((_B, D), lambda i: (i, 0)),
            pl.BlockSpec((_B, D), lambda i: (i, 0)),
            pl.BlockSpec((_B, 1), lambda i: (i, 0)),
            pl.BlockSpec((T, D, D), lambda i: (0, 0, 0)),
            pl.BlockSpec((T, D), lambda i: (0, 0)),
        ],
        out_specs=pl.BlockSpec((_B, D), lambda i: (i, 0)),
        out_shape=jax.ShapeDtypeStruct((N_P, D), jnp.float32),
    )


_tc_linear_relu = _make_tc_linear(True)
_tc_linear = _make_tc_linear(False)


def kernel(features, edge_index, node_types, W1, b1, W2, b2):
    x = jnp.pad(features.astype(jnp.float32), ((0, N_P - N), (0, 0)))
    src = jnp.pad(edge_index[0].astype(jnp.int32), (0, E_P - E))
    dst = jnp.pad(edge_index[1].astype(jnp.int32), (0, E_P - E),
                  constant_values=N)
    srcs = src.reshape(NS, BAND, CHUNK)
    dsts = dst.reshape(NS, BAND, CHUNK)
    tp = jnp.pad(node_types.astype(jnp.int32), (0, N_P - N)).reshape(N_P, 1)
    zacc = jnp.zeros((CHUNK, D), jnp.float32)
    ones = jnp.ones((CHUNK, D), jnp.float32)

    dpart = _sc_deg(dsts, zacc, ones)
    acc1 = _sc_agg(x, srcs, dsts, zacc)
    h1 = _tc_linear_relu(acc1[0], acc1[1], dpart[0], dpart[1], tp, W1, b1)
    acc2 = _sc_agg(h1, srcs, dsts, zacc)
    h2 = _tc_linear(acc2[0], acc2[1], dpart[0], dpart[1], tp, W2, b2)
    return h2[:N]


# speedup vs baseline: 1.3406x; 1.3406x over previous
import jax
import jax.numpy as jnp
from jax import lax
from jax.experimental import pallas as pl
from jax.experimental.pallas import tpu as pltpu
from jax.experimental.pallas import tpu_sc as plsc

N = 10000
D = 128
T = 8
E = 320000

NC, NS = 2, 16
NW = NC * NS
CHUNK = 64
N_P = 10112
ROWS_PER_TILE = N_P // NS
BAND = 320
E_P = NS * BAND * CHUNK
SLAB = 16
PAIRS = SLAB // 2
CH0 = 240
CH1 = BAND - CH0

_mesh = plsc.VectorSubcoreMesh(core_axis_name="c", subcore_axis_name="s")

_pieces = []
_off = 0
while _off < ROWS_PER_TILE:
    _sz = min(CHUNK, ROWS_PER_TILE - _off)
    _pieces.append((_off, _sz))
    _off += _sz


def _sc_shell(body_mid):
    def body(acc_sh, buf_v, zacc_hbm, acc_out, c, s):
        base = s * ROWS_PER_TILE
        pltpu.sync_copy(zacc_hbm, buf_v)
        for o, z in _pieces:
            pltpu.sync_copy(buf_v.at[pl.ds(0, z), :],
                            acc_sh.at[pl.ds(base + o, z), :])
        plsc.subcore_barrier()
        body_mid()
        plsc.subcore_barrier()
        for o, z in _pieces:
            pltpu.sync_copy(acc_sh.at[pl.ds(base + o, z), :],
                            buf_v.at[pl.ds(0, z), :])
            pltpu.sync_copy(buf_v.at[pl.ds(0, z), :],
                            acc_out.at[c, pl.ds(base + o, z), :])

    return body


def _make_sc_agg():
    scratch = [
        pltpu.VMEM((SLAB, CHUNK), jnp.int32),
        pltpu.VMEM((SLAB, CHUNK), jnp.int32),
        pltpu.VMEM((CHUNK, D), jnp.float32),
        pltpu.VMEM((CHUNK, D), jnp.float32),
        pltpu.VMEM_SHARED((N_P, D), jnp.float32),
        pltpu.SemaphoreType.DMA,
        pltpu.SemaphoreType.DMA,
        pltpu.SemaphoreType.DMA,
        pltpu.SemaphoreType.DMA,
    ]

    def body(x_hbm, srcs_hbm, dsts_hbm, zacc_hbm, acc_out,
             src_v, dst_v, rows0, rows1, acc_sh, sg0, sg1, ss0, ss1):
        c = lax.axis_index("c")
        s = lax.axis_index("s")

        def wait_gather(buf, sem):
            pltpu.make_async_copy(x_hbm.at[src_v.at[0]], buf, sem).wait()

        def wait_scatter(buf, sem):
            pltpu.make_async_copy(buf, acc_sh.at[dst_v.at[0]], sem).wait()

        def run_chunks(start, n_chunks):
            def pair_step(t, carry):
                a = 2 * t
                b = a + 1

                @pl.when(t > 0)
                def _():
                    wait_scatter(rows1, ss1)

                wait_gather(rows0, sg0)
                pltpu.async_copy(x_hbm.at[src_v.at[b]], rows1, sg1)
                pltpu.async_copy(rows0, acc_sh.at[dst_v.at[a]], ss0, add=True)
                wait_gather(rows1, sg1)
                wait_scatter(rows0, ss0)

                @pl.when(t < PAIRS - 1)
                def _():
                    pltpu.async_copy(x_hbm.at[src_v.at[a + 2]], rows0, sg0)

                pltpu.async_copy(rows1, acc_sh.at[dst_v.at[b]], ss1, add=True)
                return carry

            def slab_step(g, carry):
                off = start + g * SLAB
                pltpu.sync_copy(srcs_hbm.at[s, pl.ds(off, SLAB), :], src_v)
                pltpu.sync_copy(dsts_hbm.at[s, pl.ds(off, SLAB), :], dst_v)
                pltpu.async_copy(x_hbm.at[src_v.at[0]], rows0, sg0)
                lax.fori_loop(0, PAIRS, pair_step, 0)
                wait_scatter(rows1, ss1)
                return carry

            lax.fori_loop(0, n_chunks // SLAB, slab_step, 0)

        def mid():
            @pl.when(c == 0)
            def _():
                run_chunks(0, CH0)

            @pl.when(c == 1)
            def _():
                run_chunks(CH0, CH1)

        _sc_shell(mid)(acc_sh, rows0, zacc_hbm, acc_out, c, s)

    return pl.kernel(body,
                     out_type=jax.ShapeDtypeStruct((NC, N_P, D), jnp.float32),
                     mesh=_mesh, scratch_types=scratch)


def _make_sc_deg():
    GRP = 8
    scratch = [
        pltpu.VMEM((SLAB, CHUNK), jnp.int32),
        pltpu.VMEM((CHUNK, D), jnp.float32),
        pltpu.VMEM_SHARED((N_P, D), jnp.float32),
        pltpu.SemaphoreType.DMA,
    ]

    def body(dsts_hbm, zacc_hbm, ones_hbm, acc_out, dst_v, ones_v, acc_sh, sem):
        c = lax.axis_index("c")
        s = lax.axis_index("s")

        def mid():
            pltpu.sync_copy(ones_hbm, ones_v)

            def grp_step(q, carry):
                for j in range(GRP):
                    pltpu.async_copy(ones_v, acc_sh.at[dst_v.at[q * GRP + j]],
                                     sem, add=True)
                for j in range(GRP):
                    pltpu.make_async_copy(ones_v, acc_sh.at[dst_v.at[0]],
                                          sem).wait()
                return carry

            def slab_step(g, carry):
                off = c * (BAND // 2) + g * SLAB
                pltpu.sync_copy(dsts_hbm.at[s, pl.ds(off, SLAB), :], dst_v)
                lax.fori_loop(0, SLAB // GRP, grp_step, 0)
                return carry

            lax.fori_loop(0, (BAND // 2) // SLAB, slab_step, 0)

        _sc_shell(mid)(acc_sh, ones_v, zacc_hbm, acc_out, c, s)

    return pl.kernel(body,
                     out_type=jax.ShapeDtypeStruct((NC, N_P, D), jnp.float32),
                     mesh=_mesh, scratch_types=scratch)


_sc_agg = _make_sc_agg()
_sc_deg = _make_sc_deg()

_B = 2528


def _make_tc_linear(relu):
    def body(p0_ref, p1_ref, d0_ref, d1_ref, tp_ref, w_ref, b_ref, o_ref):
        deg = d0_ref[:, 0:1] + d1_ref[:, 0:1]
        deg = jnp.maximum(deg, 1.0)
        agg = (p0_ref[...] + p1_ref[...]) / deg
        t = tp_ref[...]
        acc = jnp.zeros((_B, D), jnp.float32)
        for k in range(T):
            h = jnp.dot(agg, w_ref[k], preferred_element_type=jnp.float32)
            h = h + b_ref[k][None, :]
            acc = jnp.where(t == k, h, acc)
        if relu:
            acc = jnp.maximum(acc, 0.0)
        o_ref[...] = acc

    return pl.pallas_call(
        body,
        grid=(N_P // _B,),
        in_specs=[
            pl.BlockSpec((_B, D), lambda i: (i, 0)),
            pl.BlockSpec((_B, D), lambda i: (i, 0)),
            pl.BlockSpec((_B, D), lambda i: (i, 0)),
            pl.BlockSpec((_B, D), lambda i: (i, 0)),
            pl.BlockSpec((_B, 1), lambda i: (i, 0)),
            pl.BlockSpec((T, D, D), lambda i: (0, 0, 0)),
            pl.BlockSpec((T, D), lambda i: (0, 0)),
        ],
        out_specs=pl.BlockSpec((_B, D), lambda i: (i, 0)),
        out_shape=jax.ShapeDtypeStruct((N_P, D), jnp.float32),
    )


_tc_linear_relu = _make_tc_linear(True)
_tc_linear = _make_tc_linear(False)


def kernel(features, edge_index, node_types, W1, b1, W2, b2):
    x = jnp.pad(features.astype(jnp.float32), ((0, N_P - N), (0, 0)))
    src = jnp.pad(edge_index[0].astype(jnp.int32), (0, E_P - E))
    dst = jnp.pad(edge_index[1].astype(jnp.int32), (0, E_P - E),
                  constant_values=N)
    srcs = src.reshape(NS, BAND, CHUNK)
    dsts = dst.reshape(NS, BAND, CHUNK)
    tp = jnp.pad(node_types.astype(jnp.int32), (0, N_P - N)).reshape(N_P, 1)
    zacc = jnp.zeros((CHUNK, D), jnp.float32)
    ones = jnp.ones((CHUNK, D), jnp.float32)

    dpart = _sc_deg(dsts, zacc, ones)
    acc1 = _sc_agg(x, srcs, dsts, zacc)
    h1 = _tc_linear_relu(acc1[0], acc1[1], dpart[0], dpart[1], tp, W1, b1)
    acc2 = _sc_agg(h1, srcs, dsts, zacc)
    h2 = _tc_linear(acc2[0], acc2[1], dpart[0], dpart[1], tp, W2, b2)
    return h2[:N]


# speedup vs baseline: 1.3744x; 1.0252x over previous
import jax
import jax.numpy as jnp
from jax import lax
from jax.experimental import pallas as pl
from jax.experimental.pallas import tpu as pltpu
from jax.experimental.pallas import tpu_sc as plsc

N = 10000
D = 128
T = 8
E = 320000

NC, NS = 2, 16
NW = NC * NS
CHUNK = 64
N_P = 10112
ROWS_PER_TILE = N_P // NS
BAND = 320
E_P = NS * BAND * CHUNK
SLAB = 32
PAIRS = SLAB // 2
CH0 = 224
CH1 = BAND - CH0

_mesh = plsc.VectorSubcoreMesh(core_axis_name="c", subcore_axis_name="s")

_pieces = []
_off = 0
while _off < ROWS_PER_TILE:
    _sz = min(CHUNK, ROWS_PER_TILE - _off)
    _pieces.append((_off, _sz))
    _off += _sz


def _sc_shell(body_mid):
    def body(acc_sh, buf_v, zacc_hbm, acc_out, c, s):
        base = s * ROWS_PER_TILE
        pltpu.sync_copy(zacc_hbm, buf_v)
        for o, z in _pieces:
            pltpu.sync_copy(buf_v.at[pl.ds(0, z), :],
                            acc_sh.at[pl.ds(base + o, z), :])
        plsc.subcore_barrier()
        body_mid()
        plsc.subcore_barrier()
        for o, z in _pieces:
            pltpu.sync_copy(acc_sh.at[pl.ds(base + o, z), :],
                            buf_v.at[pl.ds(0, z), :])
            pltpu.sync_copy(buf_v.at[pl.ds(0, z), :],
                            acc_out.at[c, pl.ds(base + o, z), :])

    return body


def _make_sc_agg():
    scratch = [
        pltpu.VMEM((SLAB, CHUNK), jnp.int32),
        pltpu.VMEM((SLAB, CHUNK), jnp.int32),
        pltpu.VMEM((CHUNK, D), jnp.float32),
        pltpu.VMEM((CHUNK, D), jnp.float32),
        pltpu.VMEM_SHARED((N_P, D), jnp.float32),
        pltpu.SemaphoreType.DMA,
        pltpu.SemaphoreType.DMA,
        pltpu.SemaphoreType.DMA,
        pltpu.SemaphoreType.DMA,
    ]

    def body(x_hbm, srcs_hbm, dsts_hbm, zacc_hbm, acc_out,
             src_v, dst_v, rows0, rows1, acc_sh, sg0, sg1, ss0, ss1):
        c = lax.axis_index("c")
        s = lax.axis_index("s")

        def wait_gather(buf, sem):
            pltpu.make_async_copy(x_hbm.at[src_v.at[0]], buf, sem).wait()

        def wait_scatter(buf, sem):
            pltpu.make_async_copy(buf, acc_sh.at[dst_v.at[0]], sem).wait()

        def run_chunks(start, n_chunks):
            def pair_step(t, carry):
                a = 2 * t
                b = a + 1

                @pl.when(t > 0)
                def _():
                    wait_scatter(rows1, ss1)

                wait_gather(rows0, sg0)
                pltpu.async_copy(x_hbm.at[src_v.at[b]], rows1, sg1)
                pltpu.async_copy(rows0, acc_sh.at[dst_v.at[a]], ss0, add=True)
                wait_gather(rows1, sg1)
                wait_scatter(rows0, ss0)

                @pl.when(t < PAIRS - 1)
                def _():
                    pltpu.async_copy(x_hbm.at[src_v.at[a + 2]], rows0, sg0)

                pltpu.async_copy(rows1, acc_sh.at[dst_v.at[b]], ss1, add=True)
                return carry

            def slab_step(g, carry):
                off = start + g * SLAB
                pltpu.sync_copy(srcs_hbm.at[s, pl.ds(off, SLAB), :], src_v)
                pltpu.sync_copy(dsts_hbm.at[s, pl.ds(off, SLAB), :], dst_v)
                pltpu.async_copy(x_hbm.at[src_v.at[0]], rows0, sg0)
                lax.fori_loop(0, PAIRS, pair_step, 0)
                wait_scatter(rows1, ss1)
                return carry

            lax.fori_loop(0, n_chunks // SLAB, slab_step, 0)

        def mid():
            @pl.when(c == 0)
            def _():
                run_chunks(0, CH0)

            @pl.when(c == 1)
            def _():
                run_chunks(CH0, CH1)

        _sc_shell(mid)(acc_sh, rows0, zacc_hbm, acc_out, c, s)

    return pl.kernel(body,
                     out_type=jax.ShapeDtypeStruct((NC, N_P, D), jnp.float32),
                     mesh=_mesh, scratch_types=scratch)


def _make_sc_deg():
    GRP = 8
    scratch = [
        pltpu.VMEM((SLAB, CHUNK), jnp.int32),
        pltpu.VMEM((CHUNK, D), jnp.float32),
        pltpu.VMEM_SHARED((N_P, D), jnp.float32),
        pltpu.SemaphoreType.DMA,
    ]

    def body(dsts_hbm, zacc_hbm, ones_hbm, acc_out, dst_v, ones_v, acc_sh, sem):
        c = lax.axis_index("c")
        s = lax.axis_index("s")

        def mid():
            pltpu.sync_copy(ones_hbm, ones_v)

            def grp_step(q, carry):
                for j in range(GRP):
                    pltpu.async_copy(ones_v, acc_sh.at[dst_v.at[q * GRP + j]],
                                     sem, add=True)
                for j in range(GRP):
                    pltpu.make_async_copy(ones_v, acc_sh.at[dst_v.at[0]],
                                          sem).wait()
                return carry

            def slab_step(g, carry):
                off = c * (BAND // 2) + g * SLAB
                pltpu.sync_copy(dsts_hbm.at[s, pl.ds(off, SLAB), :], dst_v)
                lax.fori_loop(0, SLAB // GRP, grp_step, 0)
                return carry

            lax.fori_loop(0, (BAND // 2) // SLAB, slab_step, 0)

        _sc_shell(mid)(acc_sh, ones_v, zacc_hbm, acc_out, c, s)

    return pl.kernel(body,
                     out_type=jax.ShapeDtypeStruct((NC, N_P, D), jnp.float32),
                     mesh=_mesh, scratch_types=scratch)


_sc_agg = _make_sc_agg()
_sc_deg = _make_sc_deg()

_B = 2528


def _make_tc_linear(relu):
    def body(p0_ref, p1_ref, d0_ref, d1_ref, tp_ref, w_ref, b_ref, o_ref):
        deg = d0_ref[:, 0:1] + d1_ref[:, 0:1]
        deg = jnp.maximum(deg, 1.0)
        agg = (p0_ref[...] + p1_ref[...]) / deg
        t = tp_ref[...]
        acc = jnp.zeros((_B, D), jnp.float32)
        for k in range(T):
            h = jnp.dot(agg, w_ref[k], preferred_element_type=jnp.float32)
            h = h + b_ref[k][None, :]
            acc = jnp.where(t == k, h, acc)
        if relu:
            acc = jnp.maximum(acc, 0.0)
        o_ref[...] = acc

    return pl.pallas_call(
        body,
        grid=(N_P // _B,),
        in_specs=[
            pl.BlockSpec((_B, D), lambda i: (i, 0)),
            pl.BlockSpec((_B, D), lambda i: (i, 0)),
            pl.BlockSpec((_B, D), lambda i: (i, 0)),
            pl.BlockSpec((_B, D), lambda i: (i, 0)),
            pl.BlockSpec((_B, 1), lambda i: (i, 0)),
            pl.BlockSpec((T, D, D), lambda i: (0, 0, 0)),
            pl.BlockSpec((T, D), lambda i: (0, 0)),
        ],
        out_specs=pl.BlockSpec((_B, D), lambda i: (i, 0)),
        out_shape=jax.ShapeDtypeStruct((N_P, D), jnp.float32),
    )


_tc_linear_relu = _make_tc_linear(True)
_tc_linear = _make_tc_linear(False)


def kernel(features, edge_index, node_types, W1, b1, W2, b2):
    x = jnp.pad(features.astype(jnp.float32), ((0, N_P - N), (0, 0)))
    src = jnp.pad(edge_index[0].astype(jnp.int32), (0, E_P - E))
    dst = jnp.pad(edge_index[1].astype(jnp.int32), (0, E_P - E),
                  constant_values=N)
    srcs = src.reshape(NS, BAND, CHUNK)
    dsts = dst.reshape(NS, BAND, CHUNK)
    tp = jnp.pad(node_types.astype(jnp.int32), (0, N_P - N)).reshape(N_P, 1)
    zacc = jnp.zeros((CHUNK, D), jnp.float32)
    ones = jnp.ones((CHUNK, D), jnp.float32)

    dpart = _sc_deg(dsts, zacc, ones)
    acc1 = _sc_agg(x, srcs, dsts, zacc)
    h1 = _tc_linear_relu(acc1[0], acc1[1], dpart[0], dpart[1], tp, W1, b1)
    acc2 = _sc_agg(h1, srcs, dsts, zacc)
    h2 = _tc_linear(acc2[0], acc2[1], dpart[0], dpart[1], tp, W2, b2)
    return h2[:N]


# speedup vs baseline: 1.3846x; 1.0074x over previous
import jax
import jax.numpy as jnp
from jax import lax
from jax.experimental import pallas as pl
from jax.experimental.pallas import tpu as pltpu
from jax.experimental.pallas import tpu_sc as plsc

N = 10000
D = 128
T = 8
E = 320000

NC, NS = 2, 16
NW = NC * NS
CHUNK = 64
N_P = 10112
ROWS_PER_TILE = N_P // NS
BAND = 320
E_P = NS * BAND * CHUNK
SLAB = 32
PAIRS = SLAB // 2
CH0 = 224
CH1 = BAND - CH0

_mesh = plsc.VectorSubcoreMesh(core_axis_name="c", subcore_axis_name="s")

_pieces = []
_off = 0
while _off < ROWS_PER_TILE:
    _sz = min(CHUNK, ROWS_PER_TILE - _off)
    _pieces.append((_off, _sz))
    _off += _sz


def _sc_shell(body_mid):
    def body(acc_sh, buf_a, buf_b, sem_a, sem_b, zacc_hbm, acc_out, c, s):
        base = s * ROWS_PER_TILE
        pltpu.sync_copy(zacc_hbm, buf_a)
        for o, z in _pieces:
            pltpu.async_copy(buf_a.at[pl.ds(0, z), :],
                             acc_sh.at[pl.ds(base + o, z), :], sem_a)
        for o, z in _pieces:
            pltpu.make_async_copy(buf_a.at[pl.ds(0, z), :],
                                  acc_sh.at[pl.ds(base + o, z), :],
                                  sem_a).wait()
        plsc.subcore_barrier()
        body_mid()
        plsc.subcore_barrier()
        np_ = len(_pieces)
        for i, (o, z) in enumerate(_pieces):
            buf, sem = (buf_a, sem_a) if i % 2 == 0 else (buf_b, sem_b)
            if i >= 2:
                po, pz = _pieces[i - 2]
                pltpu.make_async_copy(
                    buf.at[pl.ds(0, pz), :],
                    acc_out.at[c, pl.ds(base + po, pz), :], sem).wait()
            pltpu.sync_copy(acc_sh.at[pl.ds(base + o, z), :],
                            buf.at[pl.ds(0, z), :])
            pltpu.async_copy(buf.at[pl.ds(0, z), :],
                             acc_out.at[c, pl.ds(base + o, z), :], sem)
        for i in (np_ - 2, np_ - 1):
            o, z = _pieces[i]
            buf, sem = (buf_a, sem_a) if i % 2 == 0 else (buf_b, sem_b)
            pltpu.make_async_copy(buf.at[pl.ds(0, z), :],
                                  acc_out.at[c, pl.ds(base + o, z), :],
                                  sem).wait()

    return body


def _make_sc_agg():
    scratch = [
        pltpu.VMEM((SLAB, CHUNK), jnp.int32),
        pltpu.VMEM((SLAB, CHUNK), jnp.int32),
        pltpu.VMEM((CHUNK, D), jnp.float32),
        pltpu.VMEM((CHUNK, D), jnp.float32),
        pltpu.VMEM_SHARED((N_P, D), jnp.float32),
        pltpu.SemaphoreType.DMA,
        pltpu.SemaphoreType.DMA,
        pltpu.SemaphoreType.DMA,
        pltpu.SemaphoreType.DMA,
    ]

    def body(x_hbm, srcs_hbm, dsts_hbm, zacc_hbm, acc_out,
             src_v, dst_v, rows0, rows1, acc_sh, sg0, sg1, ss0, ss1):
        c = lax.axis_index("c")
        s = lax.axis_index("s")

        def wait_gather(buf, sem):
            pltpu.make_async_copy(x_hbm.at[src_v.at[0]], buf, sem).wait()

        def wait_scatter(buf, sem):
            pltpu.make_async_copy(buf, acc_sh.at[dst_v.at[0]], sem).wait()

        def run_chunks(start, n_chunks):
            def pair_step(t, carry):
                a = 2 * t
                b = a + 1

                @pl.when(t > 0)
                def _():
                    wait_scatter(rows1, ss1)

                wait_gather(rows0, sg0)
                pltpu.async_copy(x_hbm.at[src_v.at[b]], rows1, sg1)
                pltpu.async_copy(rows0, acc_sh.at[dst_v.at[a]], ss0, add=True)
                wait_gather(rows1, sg1)
                wait_scatter(rows0, ss0)

                @pl.when(t < PAIRS - 1)
                def _():
                    pltpu.async_copy(x_hbm.at[src_v.at[a + 2]], rows0, sg0)

                pltpu.async_copy(rows1, acc_sh.at[dst_v.at[b]], ss1, add=True)
                return carry

            def slab_step(g, carry):
                off = start + g * SLAB
                pltpu.sync_copy(srcs_hbm.at[s, pl.ds(off, SLAB), :], src_v)
                pltpu.sync_copy(dsts_hbm.at[s, pl.ds(off, SLAB), :], dst_v)
                pltpu.async_copy(x_hbm.at[src_v.at[0]], rows0, sg0)
                lax.fori_loop(0, PAIRS, pair_step, 0)
                wait_scatter(rows1, ss1)
                return carry

            lax.fori_loop(0, n_chunks // SLAB, slab_step, 0)

        def mid():
            @pl.when(c == 0)
            def _():
                run_chunks(0, CH0)

            @pl.when(c == 1)
            def _():
                run_chunks(CH0, CH1)

        _sc_shell(mid)(acc_sh, rows0, rows1, sg0, sg1, zacc_hbm, acc_out, c, s)

    return pl.kernel(body,
                     out_type=jax.ShapeDtypeStruct((NC, N_P, D), jnp.float32),
                     mesh=_mesh, scratch_types=scratch)


def _make_sc_deg():
    GRP = 8
    scratch = [
        pltpu.VMEM((SLAB, CHUNK), jnp.int32),
        pltpu.VMEM((CHUNK, D), jnp.float32),
        pltpu.VMEM((CHUNK, D), jnp.float32),
        pltpu.VMEM_SHARED((N_P, D), jnp.float32),
        pltpu.SemaphoreType.DMA,
        pltpu.SemaphoreType.DMA,
    ]

    def body(dsts_hbm, zacc_hbm, ones_hbm, acc_out,
             dst_v, ones_v, buf_b, acc_sh, sem, sem_b):
        c = lax.axis_index("c")
        s = lax.axis_index("s")

        def mid():
            pltpu.sync_copy(ones_hbm, ones_v)

            def grp_step(q, carry):
                for j in range(GRP):
                    pltpu.async_copy(ones_v, acc_sh.at[dst_v.at[q * GRP + j]],
                                     sem, add=True)
                for j in range(GRP):
                    pltpu.make_async_copy(ones_v, acc_sh.at[dst_v.at[0]],
                                          sem).wait()
                return carry

            def slab_step(g, carry):
                off = c * (BAND // 2) + g * SLAB
                pltpu.sync_copy(dsts_hbm.at[s, pl.ds(off, SLAB), :], dst_v)
                lax.fori_loop(0, SLAB // GRP, grp_step, 0)
                return carry

            lax.fori_loop(0, (BAND // 2) // SLAB, slab_step, 0)

        _sc_shell(mid)(acc_sh, ones_v, buf_b, sem, sem_b, zacc_hbm, acc_out, c, s)

    return pl.kernel(body,
                     out_type=jax.ShapeDtypeStruct((NC, N_P, D), jnp.float32),
                     mesh=_mesh, scratch_types=scratch)


_sc_agg = _make_sc_agg()
_sc_deg = _make_sc_deg()

_B = 2528


def _make_tc_linear(relu):
    def body(p0_ref, p1_ref, d0_ref, d1_ref, tp_ref, w_ref, b_ref, o_ref):
        deg = d0_ref[:, 0:1] + d1_ref[:, 0:1]
        deg = jnp.maximum(deg, 1.0)
        agg = (p0_ref[...] + p1_ref[...]) / deg
        t = tp_ref[...]
        acc = jnp.zeros((_B, D), jnp.float32)
        for k in range(T):
            h = jnp.dot(agg, w_ref[k], preferred_element_type=jnp.float32)
            h = h + b_ref[k][None, :]
            acc = jnp.where(t == k, h, acc)
        if relu:
            acc = jnp.maximum(acc, 0.0)
        o_ref[...] = acc

    return pl.pallas_call(
        body,
        grid=(N_P // _B,),
        in_specs=[
            pl.BlockSpec((_B, D), lambda i: (i, 0)),
            pl.BlockSpec((_B, D), lambda i: (i, 0)),
            pl.BlockSpec((_B, D), lambda i: (i, 0)),
            pl.BlockSpec((_B, D), lambda i: (i, 0)),
            pl.BlockSpec((_B, 1), lambda i: (i, 0)),
            pl.BlockSpec((T, D, D), lambda i: (0, 0, 0)),
            pl.BlockSpec((T, D), lambda i: (0, 0)),
        ],
        out_specs=pl.BlockSpec((_B, D), lambda i: (i, 0)),
        out_shape=jax.ShapeDtypeStruct((N_P, D), jnp.float32),
    )


_tc_linear_relu = _make_tc_linear(True)
_tc_linear = _make_tc_linear(False)


def kernel(features, edge_index, node_types, W1, b1, W2, b2):
    x = jnp.pad(features.astype(jnp.float32), ((0, N_P - N), (0, 0)))
    src = jnp.pad(edge_index[0].astype(jnp.int32), (0, E_P - E))
    dst = jnp.pad(edge_index[1].astype(jnp.int32), (0, E_P - E),
                  constant_values=N)
    srcs = src.reshape(NS, BAND, CHUNK)
    dsts = dst.reshape(NS, BAND, CHUNK)
    tp = jnp.pad(node_types.astype(jnp.int32), (0, N_P - N)).reshape(N_P, 1)
    zacc = jnp.zeros((CHUNK, D), jnp.float32)
    ones = jnp.ones((CHUNK, D), jnp.float32)

    dpart = _sc_deg(dsts, zacc, ones)
    acc1 = _sc_agg(x, srcs, dsts, zacc)
    h1 = _tc_linear_relu(acc1[0], acc1[1], dpart[0], dpart[1], tp, W1, b1)
    acc2 = _sc_agg(h1, srcs, dsts, zacc)
    h2 = _tc_linear(acc2[0], acc2[1], dpart[0], dpart[1], tp, W2, b2)
    return h2[:N]


# speedup vs baseline: 1.3857x; 1.0008x over previous
import jax
import jax.numpy as jnp
from jax import lax
from jax.experimental import pallas as pl
from jax.experimental.pallas import tpu as pltpu
from jax.experimental.pallas import tpu_sc as plsc

N = 10000
D = 128
T = 8
E = 320000

NC, NS = 2, 16
NW = NC * NS
CHUNK = 64
N_P = 10112
ROWS_PER_TILE = N_P // NS
BAND = 320
E_P = NS * BAND * CHUNK
SLAB = 32
PAIRS = SLAB // 2
CH0 = 224
CH1 = BAND - CH0
DW = 128

_mesh = plsc.VectorSubcoreMesh(core_axis_name="c", subcore_axis_name="s")

_pieces = []
_off = 0
while _off < ROWS_PER_TILE:
    _sz = min(CHUNK, ROWS_PER_TILE - _off)
    _pieces.append((_off, _sz))
    _off += _sz


def _sc_shell(body_mid):
    def body(acc_sh, buf_a, buf_b, sem_a, sem_b, zacc_hbm, acc_out, c, s):
        base = s * ROWS_PER_TILE
        pltpu.sync_copy(zacc_hbm, buf_a)
        for o, z in _pieces:
            pltpu.async_copy(buf_a.at[pl.ds(0, z), :],
                             acc_sh.at[pl.ds(base + o, z), :], sem_a)
        for o, z in _pieces:
            pltpu.make_async_copy(buf_a.at[pl.ds(0, z), :],
                                  acc_sh.at[pl.ds(base + o, z), :],
                                  sem_a).wait()
        plsc.subcore_barrier()
        body_mid()
        plsc.subcore_barrier()
        np_ = len(_pieces)
        for i, (o, z) in enumerate(_pieces):
            buf, sem = (buf_a, sem_a) if i % 2 == 0 else (buf_b, sem_b)
            if i >= 2:
                po, pz = _pieces[i - 2]
                pltpu.make_async_copy(
                    buf.at[pl.ds(0, pz), :],
                    acc_out.at[c, pl.ds(base + po, pz), :], sem).wait()
            pltpu.sync_copy(acc_sh.at[pl.ds(base + o, z), :],
                            buf.at[pl.ds(0, z), :])
            pltpu.async_copy(buf.at[pl.ds(0, z), :],
                             acc_out.at[c, pl.ds(base + o, z), :], sem)
        for i in (np_ - 2, np_ - 1):
            o, z = _pieces[i]
            buf, sem = (buf_a, sem_a) if i % 2 == 0 else (buf_b, sem_b)
            pltpu.make_async_copy(buf.at[pl.ds(0, z), :],
                                  acc_out.at[c, pl.ds(base + o, z), :],
                                  sem).wait()

    return body


def _make_sc_agg():
    scratch = [
        pltpu.VMEM((SLAB, CHUNK), jnp.int32),
        pltpu.VMEM((SLAB, CHUNK), jnp.int32),
        pltpu.VMEM((CHUNK, D), jnp.float32),
        pltpu.VMEM((CHUNK, D), jnp.float32),
        pltpu.VMEM_SHARED((N_P, D), jnp.float32),
        pltpu.SemaphoreType.DMA,
        pltpu.SemaphoreType.DMA,
        pltpu.SemaphoreType.DMA,
        pltpu.SemaphoreType.DMA,
    ]

    def body(x_hbm, srcs_hbm, dsts_hbm, zacc_hbm, acc_out,
             src_v, dst_v, rows0, rows1, acc_sh, sg0, sg1, ss0, ss1):
        c = lax.axis_index("c")
        s = lax.axis_index("s")

        def wait_gather(buf, sem):
            pltpu.make_async_copy(x_hbm.at[src_v.at[0]], buf, sem).wait()

        def wait_scatter(buf, sem):
            pltpu.make_async_copy(buf, acc_sh.at[dst_v.at[0]], sem).wait()

        def run_chunks(start, n_chunks):
            def pair_step(t, carry):
                a = 2 * t
                b = a + 1

                @pl.when(t > 0)
                def _():
                    wait_scatter(rows1, ss1)

                wait_gather(rows0, sg0)
                pltpu.async_copy(x_hbm.at[src_v.at[b]], rows1, sg1)
                pltpu.async_copy(rows0, acc_sh.at[dst_v.at[a]], ss0, add=True)
                wait_gather(rows1, sg1)
                wait_scatter(rows0, ss0)

                @pl.when(t < PAIRS - 1)
                def _():
                    pltpu.async_copy(x_hbm.at[src_v.at[a + 2]], rows0, sg0)

                pltpu.async_copy(rows1, acc_sh.at[dst_v.at[b]], ss1, add=True)
                return carry

            def slab_step(g, carry):
                off = start + g * SLAB
                pltpu.sync_copy(srcs_hbm.at[s, pl.ds(off, SLAB), :], src_v)
                pltpu.sync_copy(dsts_hbm.at[s, pl.ds(off, SLAB), :], dst_v)
                pltpu.async_copy(x_hbm.at[src_v.at[0]], rows0, sg0)
                lax.fori_loop(0, PAIRS, pair_step, 0)
                wait_scatter(rows1, ss1)
                return carry

            lax.fori_loop(0, n_chunks // SLAB, slab_step, 0)

        def mid():
            @pl.when(c == 0)
            def _():
                run_chunks(0, CH0)

            @pl.when(c == 1)
            def _():
                run_chunks(CH0, CH1)

        _sc_shell(mid)(acc_sh, rows0, rows1, sg0, sg1, zacc_hbm, acc_out, c, s)

    return pl.kernel(body,
                     out_type=jax.ShapeDtypeStruct((NC, N_P, D), jnp.float32),
                     mesh=_mesh, scratch_types=scratch)


def _make_sc_deg():
    GRP = 8
    scratch = [
        pltpu.VMEM((SLAB, CHUNK), jnp.int32),
        pltpu.VMEM((CHUNK, DW), jnp.float32),
        pltpu.VMEM((CHUNK, DW), jnp.float32),
        pltpu.VMEM_SHARED((N_P, DW), jnp.float32),
        pltpu.SemaphoreType.DMA,
        pltpu.SemaphoreType.DMA,
    ]

    def body(dsts_hbm, zacc_hbm, ones_hbm, acc_out,
             dst_v, ones_v, buf_b, acc_sh, sem, sem_b):
        c = lax.axis_index("c")
        s = lax.axis_index("s")

        def mid():
            pltpu.sync_copy(ones_hbm, ones_v)

            def grp_step(q, carry):
                for j in range(GRP):
                    pltpu.async_copy(ones_v, acc_sh.at[dst_v.at[q * GRP + j]],
                                     sem, add=True)
                for j in range(GRP):
                    pltpu.make_async_copy(ones_v, acc_sh.at[dst_v.at[0]],
                                          sem).wait()
                return carry

            def slab_step(g, carry):
                off = c * (BAND // 2) + g * SLAB
                pltpu.sync_copy(dsts_hbm.at[s, pl.ds(off, SLAB), :], dst_v)
                lax.fori_loop(0, SLAB // GRP, grp_step, 0)
                return carry

            lax.fori_loop(0, (BAND // 2) // SLAB, slab_step, 0)

        _sc_shell(mid)(acc_sh, ones_v, buf_b, sem, sem_b, zacc_hbm, acc_out, c, s)

    return pl.kernel(body,
                     out_type=jax.ShapeDtypeStruct((NC, N_P, DW), jnp.float32),
                     mesh=_mesh, scratch_types=scratch)


_sc_agg = _make_sc_agg()
_sc_deg = _make_sc_deg()

_B = 2528


def _make_tc_linear(relu):
    def body(p0_ref, p1_ref, d0_ref, d1_ref, tp_ref, w_ref, b_ref, o_ref):
        deg = d0_ref[:, 0:1] + d1_ref[:, 0:1]
        deg = jnp.maximum(deg, 1.0)
        agg = (p0_ref[...] + p1_ref[...]) / deg
        t = tp_ref[...]
        acc = jnp.zeros((_B, D), jnp.float32)
        for k in range(T):
            h = jnp.dot(agg, w_ref[k], preferred_element_type=jnp.float32)
            h = h + b_ref[k][None, :]
            acc = jnp.where(t == k, h, acc)
        if relu:
            acc = jnp.maximum(acc, 0.0)
        o_ref[...] = acc

    return pl.pallas_call(
        body,
        grid=(N_P // _B,),
        in_specs=[
            pl.BlockSpec((_B, D), lambda i: (i, 0)),
            pl.BlockSpec((_B, D), lambda i: (i, 0)),
            pl.BlockSpec((_B, DW), lambda i: (i, 0)),
            pl.BlockSpec((_B, DW), lambda i: (i, 0)),
            pl.BlockSpec((_B, 1), lambda i: (i, 0)),
            pl.BlockSpec((T, D, D), lambda i: (0, 0, 0)),
            pl.BlockSpec((T, D), lambda i: (0, 0)),
        ],
        out_specs=pl.BlockSpec((_B, D), lambda i: (i, 0)),
        out_shape=jax.ShapeDtypeStruct((N_P, D), jnp.float32),
    )


_tc_linear_relu = _make_tc_linear(True)
_tc_linear = _make_tc_linear(False)


def kernel(features, edge_index, node_types, W1, b1, W2, b2):
    x = jnp.pad(features.astype(jnp.float32), ((0, N_P - N), (0, 0)))
    src = jnp.pad(edge_index[0].astype(jnp.int32), (0, E_P - E))
    dst = jnp.pad(edge_index[1].astype(jnp.int32), (0, E_P - E),
                  constant_values=N)
    srcs = src.reshape(NS, BAND, CHUNK)
    dsts = dst.reshape(NS, BAND, CHUNK)
    tp = jnp.pad(node_types.astype(jnp.int32), (0, N_P - N)).reshape(N_P, 1)
    zacc = jnp.zeros((CHUNK, D), jnp.float32)
    zacc64 = jnp.zeros((CHUNK, DW), jnp.float32)
    ones64 = jnp.ones((CHUNK, DW), jnp.float32)

    dpart = _sc_deg(dsts, zacc64, ones64)
    acc1 = _sc_agg(x, srcs, dsts, zacc)
    h1 = _tc_linear_relu(acc1[0], acc1[1], dpart[0], dpart[1], tp, W1, b1)
    acc2 = _sc_agg(h1, srcs, dsts, zacc)
    h2 = _tc_linear(acc2[0], acc2[1], dpart[0], dpart[1], tp, W2, b2)
    return h2[:N]
